# baseline (speedup 1.0000x reference)
import functools

import jax
import jax.numpy as jnp
from jax import lax
from jax.experimental import pallas as pl
from jax.experimental.pallas import tpu as pltpu
from jax.experimental.pallas import tpu_sc as plsc

D = 32
NW = 32
NB = 4096
NS = 200
LB = 128
NST = NS // 8


def _make_gather_kernel():
    mesh = plsc.VectorSubcoreMesh(core_axis_name="c", subcore_axis_name="s")

    @functools.partial(
        pl.kernel,
        mesh=mesh,
        compiler_params=pltpu.CompilerParams(
            use_tc_tiling_on_sc=False, needs_layout_passes=False),
        out_type=jax.ShapeDtypeStruct((NS, D // 8, NW, 8, LB), jnp.float32),
        scratch_types=(
            pltpu.VMEM((8, LB), jnp.int32),
            pltpu.VMEM((LB, D), jnp.float32),
            pltpu.VMEM((LB, D), jnp.float32),
            pltpu.VMEM((D // 8, 8, LB), jnp.float32),
            pltpu.VMEM((D // 8, 8, LB), jnp.float32),
            pltpu.SemaphoreType.DMA,
            pltpu.SemaphoreType.DMA,
        ),
    )
    def gather_kernel(x4_hbm, w_hbm, out_hbm,
                      idx_v, gb0, gb1, ob0, ob1, gsem, osem):
        bt = lax.axis_index("s") * 2 + lax.axis_index("c")
        gbufs = (gb0, gb1)
        obufs = (ob0, ob1)
        rows = [lax.broadcasted_iota(jnp.int32, (16,), 0) + g * 16
                for g in range(8)]

        def tile_body(st, _):
            pltpu.sync_copy(x4_hbm.at[st, bt], idx_v)

            def gather(ss):
                pltpu.async_copy(w_hbm.at[idx_v.at[ss]],
                                 gbufs[ss % 2], gsem)

            def gwait(ss):
                pltpu.make_async_copy(w_hbm.at[idx_v.at[ss]],
                                      gbufs[ss % 2], gsem).wait()

            def wb(ss):
                s = st * 8 + ss
                for ct in range(D // 8):
                    pltpu.make_async_copy(
                        obufs[ss % 2].at[ct], out_hbm.at[s, ct, bt],
                        osem).start()

            def wb_wait(ss):
                s = st * 8 + ss
                for ct in range(D // 8):
                    pltpu.make_async_copy(
                        obufs[ss % 2].at[ct], out_hbm.at[s, ct, bt],
                        osem).wait()

            gather(0)
            for ss in range(8):
                if ss + 1 < 8:
                    gather(ss + 1)
                gwait(ss)
                gb, ob = gbufs[ss % 2], obufs[ss % 2]
                if ss >= 2:
                    wb_wait(ss - 2)
                for g in range(8):
                    for c in range(D):
                        vals = plsc.load_gather(
                            gb, [rows[g],
                                 jnp.full((16,), c, jnp.int32)])
                        ob[c // 8, c % 8, pl.ds(g * 16, 16)] = vals
                wb(ss)
            wb_wait(6)
            wb_wait(7)
            return 0

        lax.fori_loop(0, NST, tile_body, 0)

    return gather_kernel


_gather = _make_gather_kernel()


def kernel(input_x, weight):
    x4 = input_x.T.reshape(NST, 8, NW, LB).transpose(0, 2, 1, 3)
    out5 = _gather(x4, weight)
    return out5.transpose(2, 4, 0, 1, 3).reshape(NB, NS, D)


# speedup vs baseline: 1.0869x; 1.0869x over previous
import functools

import jax
import jax.numpy as jnp
from jax import lax
from jax.experimental import pallas as pl
from jax.experimental.pallas import tpu as pltpu
from jax.experimental.pallas import tpu_sc as plsc

D = 32
NW = 32
NB = 4096
NS = 200
LB = 128
NST = NS // 8


def _make_gather_kernel():
    mesh = plsc.VectorSubcoreMesh(core_axis_name="c", subcore_axis_name="s")

    @functools.partial(
        pl.kernel,
        mesh=mesh,
        compiler_params=pltpu.CompilerParams(
            use_tc_tiling_on_sc=False, needs_layout_passes=False),
        out_type=jax.ShapeDtypeStruct((NS, D // 8, NW, 8, LB), jnp.float32),
        scratch_types=(
            [pltpu.VMEM((NS + 8, LB), jnp.int32)]
            + [pltpu.VMEM((LB, D), jnp.float32) for _ in range(8)]
            + [pltpu.VMEM((D // 8, 8, LB), jnp.float32) for _ in range(8)]
            + [pltpu.SemaphoreType.DMA] * 3
        ),
    )
    def gather_kernel(x4_hbm, w_hbm, out_hbm, idx_all, *rest):
        gbufs = rest[:8]
        obufs = rest[8:16]
        isem, gsem, osem = rest[16], rest[17], rest[18]
        bt = lax.axis_index("s") * 2 + lax.axis_index("c")
        rows = [lax.broadcasted_iota(jnp.int32, (16,), 0) + g * 16
                for g in range(8)]
        zero16 = jnp.zeros((16,), jnp.int32)

        for st in range(NST):
            pltpu.make_async_copy(
                x4_hbm.at[st, bt], idx_all.at[pl.ds(st * 8, 8)], isem).start()
        for g in range(8):
            for r in range(NS, NS + 8):
                idx_all[r, pl.ds(g * 16, 16)] = zero16
        for st in range(NST):
            pltpu.make_async_copy(
                x4_hbm.at[st, bt], idx_all.at[pl.ds(st * 8, 8)], isem).wait()

        def gather(u, slot):
            pltpu.async_copy(w_hbm.at[idx_all.at[u]], gbufs[slot], gsem)

        def gwait(u, slot):
            pltpu.make_async_copy(
                w_hbm.at[idx_all.at[u]], gbufs[slot], gsem).wait()

        def wb_ops(t, j):
            s = t * 8 + j
            return [pltpu.make_async_copy(
                obufs[j].at[ct], out_hbm.at[s, ct, bt], osem)
                for ct in range(D // 8)]

        for j in range(8):
            gather(j, j)

        def body(t, _):
            @pl.when(t > 0)
            def _():
                for j in range(8):
                    for cp in wb_ops(t - 1, j):
                        cp.wait()

            for j in range(8):
                u = t * 8 + j
                gwait(u, j)
                gb, ob = gbufs[j], obufs[j]
                for g in range(8):
                    vals = [plsc.load_gather(
                        gb, [rows[g], jnp.full((16,), c, jnp.int32)])
                        for c in range(D)]
                    for c in range(D):
                        ob[c // 8, c % 8, pl.ds(g * 16, 16)] = vals[c]
                for cp in wb_ops(t, j):
                    cp.start()
                gather(u + 8, j)
            return 0

        lax.fori_loop(0, NST, body, 0)
        for j in range(8):
            for cp in wb_ops(NST - 1, j):
                cp.wait()
            gwait(NS + j, j)

    return gather_kernel


_gather = _make_gather_kernel()


def kernel(input_x, weight):
    x4 = input_x.T.reshape(NST, 8, NW, LB).transpose(0, 2, 1, 3)
    out5 = _gather(x4, weight)
    return out5.transpose(2, 4, 0, 1, 3).reshape(NB, NS, D)


# speedup vs baseline: 1.4713x; 1.3537x over previous
import functools

import jax
import jax.numpy as jnp
from jax import lax
from jax.experimental import pallas as pl
from jax.experimental.pallas import tpu as pltpu
from jax.experimental.pallas import tpu_sc as plsc

D = 32
NW = 32
NB = 4096
NS = 200
LB = 128
NCH = NS // 8
CH = 8 * LB


def _make_gather_kernel():
    mesh = plsc.VectorSubcoreMesh(core_axis_name="c", subcore_axis_name="s")

    @functools.partial(
        pl.kernel,
        mesh=mesh,
        compiler_params=pltpu.CompilerParams(
            use_tc_tiling_on_sc=False, needs_layout_passes=False),
        out_type=jax.ShapeDtypeStruct((NS, D // 8, NW, 8, LB), jnp.float32),
        scratch_types=(
            [pltpu.VMEM((CH,), jnp.int32) for _ in range(2)]
            + [pltpu.VMEM((CH, D), jnp.float32) for _ in range(2)]
            + [pltpu.VMEM((D // 8, 8, LB), jnp.float32) for _ in range(4)]
            + [pltpu.SemaphoreType.DMA] * 3
        ),
    )
    def gather_kernel(x3_hbm, w_hbm, out_hbm, *rest):
        ibufs = rest[:2]
        gbufs = rest[2:4]
        obufs = rest[4:8]
        isem, gsem, osem = rest[8], rest[9], rest[10]
        bt = lax.axis_index("s") * 2 + lax.axis_index("c")
        iota16 = lax.broadcasted_iota(jnp.int32, (16,), 0)

        def istart(t, slot):
            pltpu.make_async_copy(x3_hbm.at[t, bt], ibufs[slot], isem).start()

        def iwait(t, slot):
            pltpu.make_async_copy(x3_hbm.at[t, bt], ibufs[slot], isem).wait()

        def gather(slot):
            pltpu.async_copy(w_hbm.at[ibufs[slot]], gbufs[slot], gsem)

        def gwait(slot):
            pltpu.make_async_copy(
                w_hbm.at[ibufs[slot]], gbufs[slot], gsem).wait()

        def wb_ops(t, j):
            s = t * 8 + j
            return [pltpu.make_async_copy(
                obufs[j % 4].at[ct], out_hbm.at[s, ct, bt], osem)
                for ct in range(D // 8)]

        istart(0, 0)
        istart(1, 1)
        iwait(0, 0)
        gather(0)

        def consume_chunk(t, slot, first):
            gwait(slot)

            def drain_prev():
                for j in range(4, 8):
                    for cp in wb_ops(t - 1, j):
                        cp.wait()

            if first:
                pl.when(t > 0)(drain_prev)
            else:
                drain_prev()

            @pl.when(t + 2 < NCH)
            def _():
                istart(t + 2, slot)

            @pl.when(t + 1 < NCH)
            def _():
                iwait(t + 1, 1 - slot)
                gather(1 - slot)

            gb = gbufs[slot]
            for j in range(8):
                if j == 4:
                    for jj in range(4):
                        for cp in wb_ops(t, jj):
                            cp.wait()
                ob = obufs[j % 4]

                def gbody(g, _, ob=ob):
                    rvec = iota16 + (j * LB + g * 16)
                    lane = g * 16
                    for half in range(2):
                        cs = range(half * 16, half * 16 + 16)
                        vals = [plsc.load_gather(
                            gb, [rvec, jnp.full((16,), c, jnp.int32)])
                            for c in cs]
                        for c, v in zip(cs, vals):
                            ob[c // 8, c % 8, pl.ds(lane, 16)] = v
                    return 0

                lax.fori_loop(0, 8, gbody, 0)
                for cp in wb_ops(t, j):
                    cp.start()

        def body(i, _):
            consume_chunk(i * 2, 0, True)

            @pl.when(i * 2 + 1 < NCH)
            def _():
                consume_chunk(i * 2 + 1, 1, False)

            return 0

        lax.fori_loop(0, (NCH + 1) // 2, body, 0)
        for j in range(4, 8):
            for cp in wb_ops(NCH - 1, j):
                cp.wait()

    return gather_kernel


_gather = _make_gather_kernel()


def kernel(input_x, weight):
    x3 = (input_x.T.reshape(NCH, 8, NW, LB).transpose(0, 2, 1, 3)
          .reshape(NCH, NW, CH))
    out5 = _gather(x3, weight)
    return out5.transpose(2, 4, 0, 1, 3).reshape(NB, NS, D)
